# baseline (speedup 1.0000x reference)
import functools

import jax
import jax.numpy as jnp
from jax import lax
from jax.experimental import pallas as pl
from jax.experimental.pallas import tpu as pltpu
from jax.experimental.pallas import tpu_sc as plsc

NC = 2
NS = 16
NW = NC * NS
L = 16

B = 16384
D = 64
CTX = 20
VPAD = 32
CH = 128
BPW = B // NW
NCHUNK = BPW // CH


def _mean_vec(vrows, n):
    ms = []
    for k in range(D // L):
        acc = vrows[0, pl.ds(k * L, L)]
        for r in range(1, n):
            acc = acc + vrows[r, pl.ds(k * L, L)]
        ms.append(acc * (1.0 / n))
    return ms


def _dot_rows(ub, m, masks, zref, zoff):
    def group(g, carry):
        r0 = g * L
        z = jnp.zeros((L,), jnp.float32)
        for j in range(L):
            r = r0 + j
            p = ub[r, pl.ds(0, L)] * m[0]
            for k in range(1, D // L):
                p = p + ub[r, pl.ds(k * L, L)] * m[k]
            z = jnp.where(masks[j], jnp.sum(p), z)
        zref[pl.ds(zoff + r0, L)] = z
        return carry
    lax.fori_loop(0, CH // L, group, 0)


def _stage_a_body(vidx_pos, vidx_neg, posu, negu, vtab, utab,
                  zpos_out, zneg_out,
                  uidx_v, vidxp_v, vidxn_v, vrowsp, vrowsn,
                  up0, up1, up2, up3, un0, un1, un2, un3,
                  zp, zn, semv, semp, semn):
    upb = [up0, up1, up2, up3]
    unb = [un0, un1, un2, un3]
    wid = lax.axis_index("s") * NC + lax.axis_index("c")
    base = wid * BPW

    pltpu.sync_copy(vidx_pos, vidxp_v)
    pltpu.sync_copy(vidx_neg, vidxn_v)
    pltpu.sync_copy(posu.at[wid], uidx_v.at[pl.ds(0, NCHUNK)])
    pltpu.sync_copy(negu.at[wid], uidx_v.at[pl.ds(NCHUNK, NCHUNK)])

    cv1 = pltpu.async_copy(vtab.at[vidxp_v], vrowsp, semv)
    cv2 = pltpu.async_copy(vtab.at[vidxn_v], vrowsn, semv)
    cps = [pltpu.async_copy(utab.at[uidx_v.at[c]], upb[c], semp)
           for c in range(NCHUNK)]
    cns = [pltpu.async_copy(utab.at[uidx_v.at[NCHUNK + c]], unb[c], semn)
           for c in range(NCHUNK)]

    cv1.wait()
    cv2.wait()
    m_pos = _mean_vec(vrowsp, CTX)
    m_neg = _mean_vec(vrowsn, CTX)

    iota16 = lax.iota(jnp.int32, L)
    masks = [iota16 == j for j in range(L)]
    for c in range(NCHUNK):
        cps[c].wait()
        _dot_rows(upb[c], m_pos, masks, zp, c * CH)
    pltpu.sync_copy(zp, zpos_out.at[pl.ds(base, BPW)])
    for c in range(NCHUNK):
        cns[c].wait()
        _dot_rows(unb[c], m_neg, masks, zn, c * CH)
    pltpu.sync_copy(zn, zneg_out.at[pl.ds(base, BPW)])


_stage_a = functools.partial(
    pl.kernel,
    out_type=(jax.ShapeDtypeStruct((B,), jnp.float32),
              jax.ShapeDtypeStruct((B,), jnp.float32)),
    mesh=plsc.VectorSubcoreMesh(core_axis_name="c", subcore_axis_name="s",
                                num_cores=NC, num_subcores=NS),
    compiler_params=pltpu.CompilerParams(needs_layout_passes=False,
                                         use_tc_tiling_on_sc=False),
    scratch_types=[
        pltpu.VMEM((2 * NCHUNK, CH), jnp.int32),
        pltpu.VMEM((VPAD,), jnp.int32),
        pltpu.VMEM((VPAD,), jnp.int32),
        pltpu.VMEM((VPAD, D), jnp.float32),
        pltpu.VMEM((VPAD, D), jnp.float32),
        pltpu.VMEM((CH, D), jnp.float32),
        pltpu.VMEM((CH, D), jnp.float32),
        pltpu.VMEM((CH, D), jnp.float32),
        pltpu.VMEM((CH, D), jnp.float32),
        pltpu.VMEM((CH, D), jnp.float32),
        pltpu.VMEM((CH, D), jnp.float32),
        pltpu.VMEM((CH, D), jnp.float32),
        pltpu.VMEM((CH, D), jnp.float32),
        pltpu.VMEM((BPW,), jnp.float32),
        pltpu.VMEM((BPW,), jnp.float32),
        pltpu.SemaphoreType.DMA,
        pltpu.SemaphoreType.DMA,
        pltpu.SemaphoreType.DMA,
    ],
)(_stage_a_body)


def _loss_body(pz_ref, nz_ref, o_ref):
    def logsig(x):
        return jnp.minimum(x, 0.0) - jnp.log1p(jnp.exp(-jnp.abs(x)))
    total = -(jnp.sum(logsig(pz_ref[...])) + jnp.sum(logsig(-nz_ref[...])))
    o_ref[...] = jnp.reshape(total, (1, 1))


_loss = pl.pallas_call(
    _loss_body,
    out_shape=jax.ShapeDtypeStruct((1, 1), jnp.float32),
)


def kernel(pos_v, pos_u, neg_v, neg_u, v_table, u_table):
    vidx_pos = jnp.pad(pos_v[-1], (0, VPAD - CTX))
    vidx_neg = jnp.pad(neg_v[-1], (0, VPAD - CTX))
    posu = pos_u.reshape(NW, NCHUNK, CH)
    negu = neg_u.reshape(NW, NCHUNK, CH)
    zp, zn = _stage_a(vidx_pos, vidx_neg, posu, negu, v_table, u_table)
    out = _loss(zp.reshape(B // 128, 128), zn.reshape(B // 128, 128))
    return out[0, 0]


# speedup vs baseline: 1.7968x; 1.7968x over previous
import functools

import jax
import jax.numpy as jnp
from jax import lax
from jax.experimental import pallas as pl
from jax.experimental.pallas import tpu as pltpu
from jax.experimental.pallas import tpu_sc as plsc

NC = 2
NS = 16
NW = NC * NS
L = 16

B = 16384
D = 64
CTX = 20
CH = 128
BPW = B // NW
NCHUNK = BPW // CH


def _mean_body(idx_ref, blk_ref, o_ref):
    j = pl.program_id(0)
    c = idx_ref[j] % 128
    lane = lax.broadcasted_iota(jnp.int32, (D, 128), 1)
    col = jnp.sum(jnp.where(lane == c, blk_ref[...], 0.0), axis=1,
                  keepdims=True)

    @pl.when(j == 0)
    def _():
        o_ref[...] = jnp.zeros_like(o_ref)
    rowmask = lax.broadcasted_iota(jnp.int32, (2, D), 0) == j // CTX
    o_ref[...] += jnp.where(rowmask, col.reshape(1, D), 0.0) * (1.0 / CTX)


_mean = pl.pallas_call(
    _mean_body,
    grid_spec=pltpu.PrefetchScalarGridSpec(
        num_scalar_prefetch=1,
        grid=(2 * CTX,),
        in_specs=[pl.BlockSpec((D, 128),
                               lambda j, idx_ref: (0, idx_ref[j] // 128))],
        out_specs=pl.BlockSpec((2, D), lambda j, idx_ref: (0, 0)),
    ),
    out_shape=jax.ShapeDtypeStruct((2, D), jnp.float32),
)


def _dot_rows(ub, m, masks, zref, zoff):
    def group(g, carry):
        r0 = g * L
        z = jnp.zeros((L,), jnp.float32)
        for j in range(L):
            r = r0 + j
            p = ub[r, pl.ds(0, L)] * m[0]
            for k in range(1, D // L):
                p = p + ub[r, pl.ds(k * L, L)] * m[k]
            z = jnp.where(masks[j], jnp.sum(p), z)
        zref[pl.ds(zoff + r0, L)] = z
        return carry
    lax.fori_loop(0, CH // L, group, 0)


def _stage_a_body(means, posu, negu, utab,
                  zpos_out, zneg_out,
                  uidx_v, mv, up0, up1, up2, up3, un0, un1, un2, un3,
                  zp, zn, semp, semn):
    upb = [up0, up1, up2, up3]
    unb = [un0, un1, un2, un3]
    wid = lax.axis_index("s") * NC + lax.axis_index("c")
    base = wid * BPW

    pltpu.sync_copy(means, mv)
    pltpu.sync_copy(posu.at[wid], uidx_v.at[pl.ds(0, NCHUNK)])
    pltpu.sync_copy(negu.at[wid], uidx_v.at[pl.ds(NCHUNK, NCHUNK)])

    cps = [pltpu.async_copy(utab.at[uidx_v.at[c]], upb[c], semp)
           for c in range(NCHUNK)]
    cns = [pltpu.async_copy(utab.at[uidx_v.at[NCHUNK + c]], unb[c], semn)
           for c in range(NCHUNK)]

    m_pos = [mv[0, pl.ds(k * L, L)] for k in range(D // L)]
    m_neg = [mv[1, pl.ds(k * L, L)] for k in range(D // L)]

    iota16 = lax.iota(jnp.int32, L)
    masks = [iota16 == j for j in range(L)]
    for c in range(NCHUNK):
        cps[c].wait()
        _dot_rows(upb[c], m_pos, masks, zp, c * CH)
    pltpu.sync_copy(zp, zpos_out.at[pl.ds(base, BPW)])
    for c in range(NCHUNK):
        cns[c].wait()
        _dot_rows(unb[c], m_neg, masks, zn, c * CH)
    pltpu.sync_copy(zn, zneg_out.at[pl.ds(base, BPW)])


_stage_a = functools.partial(
    pl.kernel,
    out_type=(jax.ShapeDtypeStruct((B,), jnp.float32),
              jax.ShapeDtypeStruct((B,), jnp.float32)),
    mesh=plsc.VectorSubcoreMesh(core_axis_name="c", subcore_axis_name="s",
                                num_cores=NC, num_subcores=NS),
    compiler_params=pltpu.CompilerParams(needs_layout_passes=False,
                                         use_tc_tiling_on_sc=False),
    scratch_types=[
        pltpu.VMEM((2 * NCHUNK, CH), jnp.int32),
        pltpu.VMEM((2, D), jnp.float32),
        pltpu.VMEM((CH, D), jnp.float32),
        pltpu.VMEM((CH, D), jnp.float32),
        pltpu.VMEM((CH, D), jnp.float32),
        pltpu.VMEM((CH, D), jnp.float32),
        pltpu.VMEM((CH, D), jnp.float32),
        pltpu.VMEM((CH, D), jnp.float32),
        pltpu.VMEM((CH, D), jnp.float32),
        pltpu.VMEM((CH, D), jnp.float32),
        pltpu.VMEM((BPW,), jnp.float32),
        pltpu.VMEM((BPW,), jnp.float32),
        pltpu.SemaphoreType.DMA,
        pltpu.SemaphoreType.DMA,
    ],
)(_stage_a_body)


def _loss_body(pz_ref, nz_ref, o_ref):
    def logsig(x):
        return jnp.minimum(x, 0.0) - jnp.log1p(jnp.exp(-jnp.abs(x)))
    total = -(jnp.sum(logsig(pz_ref[...])) + jnp.sum(logsig(-nz_ref[...])))
    o_ref[...] = jnp.reshape(total, (1, 1))


_loss = pl.pallas_call(
    _loss_body,
    out_shape=jax.ShapeDtypeStruct((1, 1), jnp.float32),
)


def kernel(pos_v, pos_u, neg_v, neg_u, v_table, u_table):
    vidx = jnp.concatenate([pos_v[-1], neg_v[-1]])
    means = _mean(vidx, v_table.T)
    posu = pos_u.reshape(NW, NCHUNK, CH)
    negu = neg_u.reshape(NW, NCHUNK, CH)
    zp, zn = _stage_a(means, posu, negu, u_table)
    out = _loss(zp.reshape(B // 128, 128), zn.reshape(B // 128, 128))
    return out[0, 0]
